# baseline (speedup 1.0000x reference)
import functools

import jax
import jax.numpy as jnp
from jax import lax
from jax.experimental import pallas as pl
from jax.experimental.pallas import tpu as pltpu
from jax.experimental.pallas import tpu_sc as plsc

N_NODES = 4096
N_EDGES = 65536
B = 64
NPG = 64
D_IN = 128
D_EDGE = 16
HID = 128

NC = 2
NS = 16
NW = NC * NS
L = 16


_EB = 2048
_XB = N_NODES // (N_EDGES // _EB)


def _edge1_body(ea_ref, bw_ref, bb_ref, w1_ref, b1_ref, w2_ref, b2_ref,
                x_ref, aw_ref, ab_ref, ee_ref, h0_ref):
    e0 = jnp.dot(ea_ref[...], bw_ref[...],
                 preferred_element_type=jnp.float32) + bb_ref[...]
    t = jnp.maximum(jnp.dot(e0, w1_ref[...],
                            preferred_element_type=jnp.float32) + b1_ref[...], 0.0)
    ee_ref[...] = jnp.dot(t, w2_ref[...],
                          preferred_element_type=jnp.float32) + b2_ref[...]
    h0_ref[...] = jnp.dot(x_ref[...], aw_ref[...],
                          preferred_element_type=jnp.float32) + ab_ref[...]


def _edge1_and_atom(edge_attr, x, params):
    lp1 = params['layers'][0]
    full = lambda shape: pl.BlockSpec(shape, lambda i: (0,) * len(shape))
    row = lambda d: pl.BlockSpec((_EB, d), lambda i: (i, 0))
    return pl.pallas_call(
        _edge1_body,
        grid=(N_EDGES // _EB,),
        in_specs=[
            row(D_EDGE),
            full((D_EDGE, HID)), full((1, HID)),
            full((HID, HID)), full((1, HID)), full((HID, HID)), full((1, HID)),
            pl.BlockSpec((_XB, D_IN), lambda i: (i, 0)),
            full((D_IN, HID)), full((1, HID)),
        ],
        out_specs=[row(HID), pl.BlockSpec((_XB, HID), lambda i: (i, 0))],
        out_shape=[jax.ShapeDtypeStruct((N_EDGES, HID), jnp.float32),
                   jax.ShapeDtypeStruct((N_NODES, HID), jnp.float32)],
    )(edge_attr,
      params['bond_W'], params['bond_b'][None, :],
      lp1['be_W1'], lp1['be_b1'][None, :],
      lp1['be_W2'], lp1['be_b2'][None, :],
      x, params['atom_W'], params['atom_b'][None, :])


def _edge2_body(ea_ref, bw_ref, bb_ref, w1_ref, b1_ref, w2_ref, b2_ref,
                ee_ref):
    e0 = jnp.dot(ea_ref[...], bw_ref[...],
                 preferred_element_type=jnp.float32) + bb_ref[...]
    t = jnp.maximum(jnp.dot(e0, w1_ref[...],
                            preferred_element_type=jnp.float32) + b1_ref[...], 0.0)
    ee_ref[...] = jnp.dot(t, w2_ref[...],
                          preferred_element_type=jnp.float32) + b2_ref[...]


def _edge2_mlp(edge_attr, params):
    lp2 = params['layers'][1]
    full = lambda shape: pl.BlockSpec(shape, lambda i: (0,) * len(shape))
    row = lambda d: pl.BlockSpec((_EB, d), lambda i: (i, 0))
    return pl.pallas_call(
        _edge2_body,
        grid=(N_EDGES // _EB,),
        in_specs=[
            row(D_EDGE),
            full((D_EDGE, HID)), full((1, HID)),
            full((HID, HID)), full((1, HID)), full((HID, HID)), full((1, HID)),
        ],
        out_specs=row(HID),
        out_shape=jax.ShapeDtypeStruct((N_EDGES, HID), jnp.float32),
    )(edge_attr, params['bond_W'], params['bond_b'][None, :],
      lp2['be_W1'], lp2['be_b1'][None, :],
      lp2['be_W2'], lp2['be_b2'][None, :])


_ECH = 128
_EPW = N_EDGES // NW
_NCHUNK = _EPW // _ECH
_RPT = N_NODES // NS
_NPAIR = B * NPG * NPG
_WIN = _NPAIR // NW
_ACH = 2048

_sc_mesh = plsc.VectorSubcoreMesh(core_axis_name="c", subcore_axis_name="s")


def _make_aggr(do_adj):
    out_type = [jax.ShapeDtypeStruct((NC, N_NODES, HID), jnp.float32)]
    scratch = [
        pltpu.VMEM((_NCHUNK, _ECH), jnp.int32),
        pltpu.VMEM((_NCHUNK, _ECH), jnp.int32),
        pltpu.VMEM((2, _ECH, HID), jnp.float32),
        pltpu.VMEM((2, _ECH, HID), jnp.float32),
        pltpu.VMEM_SHARED((N_NODES, HID), jnp.float32),
        pltpu.SemaphoreType.DMA((2,)),
        pltpu.SemaphoreType.DMA((2,)),
    ]
    if do_adj:
        out_type.append(jax.ShapeDtypeStruct((_NPAIR,), jnp.float32))
        scratch += [
            pltpu.VMEM((_WIN,), jnp.float32),
            pltpu.VMEM((2, _NCHUNK, _ECH), jnp.int32),
            pltpu.VMEM((2, _NCHUNK, _ECH), jnp.int32),
            pltpu.SemaphoreType.DMA((2,)),
            pltpu.SemaphoreType.DMA((2,)),
        ]

    @functools.partial(
        pl.kernel,
        out_type=out_type,
        mesh=_sc_mesh,
        scratch_types=scratch,
        compiler_params=pltpu.CompilerParams(needs_layout_passes=False),
    )
    def _kernel(h_hbm, ee_hbm, ei_hbm, zeros_hbm, out_hbm, *rest):
        if do_adj:
            (adj_hbm, srcs, dsts, rows2, ee2, acc, gsem, esem,
             win, asv, adv, assem, adsem) = rest
        else:
            srcs, dsts, rows2, ee2, acc, gsem, esem = rest
        c = lax.axis_index("c")
        s = lax.axis_index("s")
        wid = c * NS + s
        ebase = wid * _EPW
        pltpu.sync_copy(zeros_hbm.at[pl.ds(s * _RPT, _RPT)],
                        acc.at[pl.ds(s * _RPT, _RPT)])
        pltpu.sync_copy(ei_hbm.at[0, wid], srcs)
        pltpu.sync_copy(ei_hbm.at[1, wid], dsts)

        def _start(j, b):
            pltpu.async_copy(h_hbm.at[srcs.at[j]], rows2.at[b], gsem.at[b])
            pltpu.async_copy(ee_hbm.at[pl.ds(ebase + j * _ECH, _ECH), :],
                             ee2.at[b], esem.at[b])

        if do_adj:
            wbase = wid * _WIN
            ones = jnp.full((L,), 1.0, jnp.float32)

            def _adj_start(t, ab):
                pltpu.async_copy(ei_hbm.at[0, t], asv.at[ab], assem.at[ab])
                pltpu.async_copy(ei_hbm.at[1, t], adv.at[ab], adsem.at[ab])

            def _adj_step(t, ab):
                @pl.when(t + 1 < NW)
                def _():
                    _adj_start(t + 1, 1 - ab)

                pltpu.make_async_copy(ei_hbm.at[0, t], asv.at[ab],
                                      assem.at[ab]).wait()
                pltpu.make_async_copy(ei_hbm.at[1, t], adv.at[ab],
                                      adsem.at[ab]).wait()

                @pl.loop(0, _NCHUNK, unroll=2)
                def _inner(rr):
                    for cb in range(_ECH // L):
                        sl = pl.ds(cb * L, L)
                        sv = asv[ab, rr, sl]
                        dv = adv[ab, rr, sl]
                        pos = sv * NPG + (dv & (NPG - 1)) - wbase
                        m = ((sv >> 6) == (dv >> 6)) & (pos >= 0) & (pos < _WIN)
                        plsc.store_scatter(win, [pos], ones, mask=m)

            _adj_start(0, 0)

            @pl.loop(0, _WIN // L, unroll=8)
            def _z(i):
                win[pl.ds(i * L, L)] = jnp.zeros((L,), jnp.float32)

        plsc.subcore_barrier()
        _start(0, 0)

        @pl.loop(0, _NCHUNK, step=2)
        def _chunk(j0):
            for b in range(2):
                j = j0 + b

                @pl.when(j + 1 < _NCHUNK)
                def _():
                    _start(j + 1, 1 - b)

                if do_adj:
                    _adj_step(2 * j, 0)
                    _adj_step(2 * j + 1, 1)

                pltpu.make_async_copy(h_hbm.at[srcs.at[j]], rows2.at[b],
                                      gsem.at[b]).wait()
                pltpu.make_async_copy(
                    ee_hbm.at[pl.ds(ebase + j * _ECH, _ECH), :],
                    ee2.at[b], esem.at[b]).wait()

                @pl.loop(0, _ECH, unroll=2)
                def _row(r):
                    for cb in range(HID // L):
                        sl = pl.ds(cb * L, L)
                        rows2[b, r, sl] = jnp.maximum(
                            rows2[b, r, sl] + ee2[b, r, sl], 0.0)

                pltpu.sync_copy(rows2.at[b], acc.at[dsts.at[j]], add=True)

        if do_adj:
            pltpu.sync_copy(win, adj_hbm.at[pl.ds(wbase, _WIN)])
        plsc.subcore_barrier()
        pltpu.sync_copy(acc.at[pl.ds(s * _RPT, _RPT)],
                        out_hbm.at[c, pl.ds(s * _RPT, _RPT)])

    return _kernel


_gine_aggr_adj_kernel = _make_aggr(True)
_gine_aggr_kernel = _make_aggr(False)


def _node_body(h_ref, p_ref, eps_ref, w1_ref, b1_ref, w2_ref, b2_ref,
               g_ref, be_ref, o_ref):
    z = (1.0 + eps_ref[0, 0]) * h_ref[...] + p_ref[0] + p_ref[1]
    z = jnp.maximum(jnp.dot(z, w1_ref[...],
                            preferred_element_type=jnp.float32) + b1_ref[...], 0.0)
    z = jnp.dot(z, w2_ref[...], preferred_element_type=jnp.float32) + b2_ref[...]
    mu = jnp.mean(z, axis=0, keepdims=True)
    var = jnp.mean(jnp.square(z - mu), axis=0, keepdims=True)
    z = (z - mu) * jax.lax.rsqrt(var + 1e-5) * g_ref[...] + be_ref[...]
    o_ref[...] = jnp.maximum(z, 0.0)


def _node_update(h, partials, lp):
    return pl.pallas_call(
        _node_body,
        out_shape=jax.ShapeDtypeStruct((N_NODES, HID), jnp.float32),
    )(h, partials, lp['eps'][None, None],
      lp['nn_W1'], lp['nn_b1'][None, :], lp['nn_W2'], lp['nn_b2'][None, :],
      lp['bn_gamma'][None, :], lp['bn_beta'][None, :])


def _node2_body(h_ref, p_ref, eps_ref, w1_ref, b1_ref, w2_ref, b2_ref,
                g_ref, be_ref, wa_ref, wb_ref, mb1_ref, a_ref, bm_ref):
    z = (1.0 + eps_ref[0, 0]) * h_ref[...] + p_ref[0] + p_ref[1]
    z = jnp.maximum(jnp.dot(z, w1_ref[...],
                            preferred_element_type=jnp.float32) + b1_ref[...], 0.0)
    z = jnp.dot(z, w2_ref[...], preferred_element_type=jnp.float32) + b2_ref[...]
    mu = jnp.mean(z, axis=0, keepdims=True)
    var = jnp.mean(jnp.square(z - mu), axis=0, keepdims=True)
    z = (z - mu) * jax.lax.rsqrt(var + 1e-5) * g_ref[...] + be_ref[...]
    h2 = jnp.maximum(z, 0.0)
    a_ref[...] = jnp.dot(h2, wa_ref[...],
                         preferred_element_type=jnp.float32) + mb1_ref[...]
    bm_ref[...] = jnp.dot(h2, wb_ref[...], preferred_element_type=jnp.float32)


def _node2_and_pair_ab(h, partials, lp, params):
    w1 = params['mlp_W1']
    return pl.pallas_call(
        _node2_body,
        out_shape=[jax.ShapeDtypeStruct((N_NODES, HID), jnp.float32)] * 2,
    )(h, partials, lp['eps'][None, None],
      lp['nn_W1'], lp['nn_b1'][None, :], lp['nn_W2'], lp['nn_b2'][None, :],
      lp['bn_gamma'][None, :], lp['bn_beta'][None, :],
      w1[:HID], w1[HID:2 * HID], params['mlp_b1'][None, :])


_GPB = 4


def _pair_body(a_ref, b_ref, adj_ref, w_ref, w2_ref, b2_ref, o_ref):
    w = w_ref[0][None, None, :].astype(jnp.bfloat16)
    w2 = w2_ref[...].astype(jnp.bfloat16)
    for g in range(_GPB):
        a = a_ref[g].astype(jnp.bfloat16)
        b = b_ref[g].astype(jnp.bfloat16)
        adj = adj_ref[g].astype(jnp.bfloat16)
        t = a[:, None, :] + b[None, :, :] + adj[:, :, None] * w
        t = jnp.maximum(t, 0.0).reshape(NPG * NPG, HID)
        o_ref[g] = (jnp.dot(t, w2, preferred_element_type=jnp.float32)
                    + b2_ref[0, 0]).reshape(NPG, NPG)


def _pair_stage(a, bmat, adjflat, params):
    out = pl.pallas_call(
        _pair_body,
        grid=(B // _GPB,),
        in_specs=[
            pl.BlockSpec((_GPB, NPG, HID), lambda i: (i, 0, 0)),
            pl.BlockSpec((_GPB, NPG, HID), lambda i: (i, 0, 0)),
            pl.BlockSpec((_GPB, NPG, NPG), lambda i: (i, 0, 0)),
            pl.BlockSpec((1, HID), lambda i: (0, 0)),
            pl.BlockSpec((HID, 1), lambda i: (0, 0)),
            pl.BlockSpec((1, 1), lambda i: (0, 0)),
        ],
        out_specs=pl.BlockSpec((_GPB, NPG, NPG), lambda i: (i, 0, 0)),
        out_shape=jax.ShapeDtypeStruct((B, NPG, NPG), jnp.float32),
    )(a.reshape(B, NPG, HID), bmat.reshape(B, NPG, HID),
      adjflat.reshape(B, NPG, NPG),
      params['mlp_W1'][2 * HID][None, :], params['mlp_W2'],
      params['mlp_b2'][None, :])
    return out.reshape(B * NPG * NPG, 1)


@jax.jit
def kernel(x, edge_index, edge_attr, params):
    lp1, lp2 = params['layers']
    zeros = jnp.zeros((N_NODES, HID), jnp.float32)
    ei4 = edge_index.reshape(2, NW, _NCHUNK, _ECH)
    ee1, h0 = _edge1_and_atom(edge_attr, x, params)
    partials1, adjflat = _gine_aggr_adj_kernel(h0, ee1, ei4, zeros)
    ee2 = _edge2_mlp(edge_attr, params)
    h1 = _node_update(h0, partials1, lp1)
    partials2, = _gine_aggr_kernel(h1, ee2, ei4, zeros)
    a, bmat = _node2_and_pair_ab(h1, partials2, lp2, params)
    return _pair_stage(a, bmat, adjflat, params)


# speedup vs baseline: 1.2183x; 1.2183x over previous
import functools

import jax
import jax.numpy as jnp
from jax import lax
from jax.experimental import pallas as pl
from jax.experimental.pallas import tpu as pltpu
from jax.experimental.pallas import tpu_sc as plsc

N_NODES = 4096
N_EDGES = 65536
B = 64
NPG = 64
D_IN = 128
D_EDGE = 16
HID = 128

NC = 2
NS = 16
NW = NC * NS
L = 16


_EB = 2048
_XB = N_NODES // (N_EDGES // _EB)


def _edge1_body(ea_ref, bw_ref, bb_ref, w1_ref, b1_ref, w2_ref, b2_ref,
                x_ref, aw_ref, ab_ref, ee_ref, h0_ref):
    e0 = jnp.dot(ea_ref[...], bw_ref[...],
                 preferred_element_type=jnp.float32) + bb_ref[...]
    t = jnp.maximum(jnp.dot(e0, w1_ref[...],
                            preferred_element_type=jnp.float32) + b1_ref[...], 0.0)
    ee_ref[...] = jnp.dot(t, w2_ref[...],
                          preferred_element_type=jnp.float32) + b2_ref[...]
    h0_ref[...] = jnp.dot(x_ref[...], aw_ref[...],
                          preferred_element_type=jnp.float32) + ab_ref[...]


def _edge1_and_atom(edge_attr, x, params):
    lp1 = params['layers'][0]
    full = lambda shape: pl.BlockSpec(shape, lambda i: (0,) * len(shape))
    row = lambda d: pl.BlockSpec((_EB, d), lambda i: (i, 0))
    return pl.pallas_call(
        _edge1_body,
        grid=(N_EDGES // _EB,),
        in_specs=[
            row(D_EDGE),
            full((D_EDGE, HID)), full((1, HID)),
            full((HID, HID)), full((1, HID)), full((HID, HID)), full((1, HID)),
            pl.BlockSpec((_XB, D_IN), lambda i: (i, 0)),
            full((D_IN, HID)), full((1, HID)),
        ],
        out_specs=[row(HID), pl.BlockSpec((_XB, HID), lambda i: (i, 0))],
        out_shape=[jax.ShapeDtypeStruct((N_EDGES, HID), jnp.float32),
                   jax.ShapeDtypeStruct((N_NODES, HID), jnp.float32)],
    )(edge_attr,
      params['bond_W'], params['bond_b'][None, :],
      lp1['be_W1'], lp1['be_b1'][None, :],
      lp1['be_W2'], lp1['be_b2'][None, :],
      x, params['atom_W'], params['atom_b'][None, :])


def _edge2_body(ea_ref, bw_ref, bb_ref, w1_ref, b1_ref, w2_ref, b2_ref,
                ee_ref):
    e0 = jnp.dot(ea_ref[...], bw_ref[...],
                 preferred_element_type=jnp.float32) + bb_ref[...]
    t = jnp.maximum(jnp.dot(e0, w1_ref[...],
                            preferred_element_type=jnp.float32) + b1_ref[...], 0.0)
    ee_ref[...] = jnp.dot(t, w2_ref[...],
                          preferred_element_type=jnp.float32) + b2_ref[...]


def _edge2_mlp(edge_attr, params):
    lp2 = params['layers'][1]
    full = lambda shape: pl.BlockSpec(shape, lambda i: (0,) * len(shape))
    row = lambda d: pl.BlockSpec((_EB, d), lambda i: (i, 0))
    return pl.pallas_call(
        _edge2_body,
        grid=(N_EDGES // _EB,),
        in_specs=[
            row(D_EDGE),
            full((D_EDGE, HID)), full((1, HID)),
            full((HID, HID)), full((1, HID)), full((HID, HID)), full((1, HID)),
        ],
        out_specs=row(HID),
        out_shape=jax.ShapeDtypeStruct((N_EDGES, HID), jnp.float32),
    )(edge_attr, params['bond_W'], params['bond_b'][None, :],
      lp2['be_W1'], lp2['be_b1'][None, :],
      lp2['be_W2'], lp2['be_b2'][None, :])


_ECH = 128
_EPW = N_EDGES // NW
_NCHUNK = _EPW // _ECH
_RPT = N_NODES // NS
_NPAIR = B * NPG * NPG
_WIN = _NPAIR // NW
_ACH = 2048

_sc_mesh = plsc.VectorSubcoreMesh(core_axis_name="c", subcore_axis_name="s")


def _make_aggr(do_adj):
    out_type = [jax.ShapeDtypeStruct((NC, N_NODES, HID), jnp.float32)]
    scratch = [
        pltpu.VMEM((_NCHUNK, _ECH), jnp.int32),
        pltpu.VMEM((_NCHUNK, _ECH), jnp.int32),
        pltpu.VMEM((2, _ECH, HID), jnp.float32),
        pltpu.VMEM((2, _ECH, HID), jnp.float32),
        pltpu.VMEM_SHARED((N_NODES, HID), jnp.float32),
        pltpu.SemaphoreType.DMA((2,)),
        pltpu.SemaphoreType.DMA((2,)),
        pltpu.SemaphoreType.DMA((2,)),
    ]
    if do_adj:
        out_type.append(jax.ShapeDtypeStruct((_NPAIR,), jnp.float32))
        scratch += [
            pltpu.VMEM((_WIN,), jnp.float32),
            pltpu.VMEM((2, _NCHUNK, _ECH), jnp.int32),
            pltpu.VMEM((2, _NCHUNK, _ECH), jnp.int32),
            pltpu.SemaphoreType.DMA((2,)),
            pltpu.SemaphoreType.DMA((2,)),
        ]

    @functools.partial(
        pl.kernel,
        out_type=out_type,
        mesh=_sc_mesh,
        scratch_types=scratch,
        compiler_params=pltpu.CompilerParams(needs_layout_passes=False),
    )
    def _kernel(h_hbm, ee_hbm, ei_hbm, zeros_hbm, out_hbm, *rest):
        if do_adj:
            (adj_hbm, srcs, dsts, rows2, ee2, acc, gsem, esem, ssem,
             win, asv, adv, assem, adsem) = rest
        else:
            srcs, dsts, rows2, ee2, acc, gsem, esem, ssem = rest
        c = lax.axis_index("c")
        s = lax.axis_index("s")
        wid = c * NS + s
        ebase = wid * _EPW
        pltpu.sync_copy(zeros_hbm.at[pl.ds(s * _RPT, _RPT)],
                        acc.at[pl.ds(s * _RPT, _RPT)])
        pltpu.sync_copy(ei_hbm.at[0, wid], srcs)
        pltpu.sync_copy(ei_hbm.at[1, wid], dsts)

        def _start(j, b):
            pltpu.async_copy(h_hbm.at[srcs.at[j]], rows2.at[b], gsem.at[b])
            pltpu.async_copy(ee_hbm.at[pl.ds(ebase + j * _ECH, _ECH), :],
                             ee2.at[b], esem.at[b])

        if do_adj:
            wbase = wid * _WIN
            ones = jnp.full((L,), 1.0, jnp.float32)

            def _adj_start(t, ab):
                pltpu.async_copy(ei_hbm.at[0, t], asv.at[ab], assem.at[ab])
                pltpu.async_copy(ei_hbm.at[1, t], adv.at[ab], adsem.at[ab])

            def _adj_step(t, ab):
                @pl.when(t + 1 < NW)
                def _():
                    _adj_start(t + 1, 1 - ab)

                pltpu.make_async_copy(ei_hbm.at[0, t], asv.at[ab],
                                      assem.at[ab]).wait()
                pltpu.make_async_copy(ei_hbm.at[1, t], adv.at[ab],
                                      adsem.at[ab]).wait()

                @pl.loop(0, _NCHUNK, unroll=2)
                def _inner(rr):
                    for cb in range(_ECH // L):
                        sl = pl.ds(cb * L, L)
                        sv = asv[ab, rr, sl]
                        dv = adv[ab, rr, sl]
                        pos = sv * NPG + (dv & (NPG - 1)) - wbase
                        m = ((sv >> 6) == (dv >> 6)) & (pos >= 0) & (pos < _WIN)
                        plsc.store_scatter(win, [pos], ones, mask=m)

            _adj_start(0, 0)

            @pl.loop(0, _WIN // L, unroll=8)
            def _z(i):
                win[pl.ds(i * L, L)] = jnp.zeros((L,), jnp.float32)

        plsc.subcore_barrier()
        _start(0, 0)

        @pl.loop(0, _NCHUNK, step=2)
        def _chunk(j0):
            for b in range(2):
                j = j0 + b

                @pl.when((j >= 1) & (j + 1 < _NCHUNK))
                def _():
                    pltpu.make_async_copy(rows2.at[1 - b],
                                          acc.at[dsts.at[j - 1]],
                                          ssem.at[1 - b]).wait()

                @pl.when(j + 1 < _NCHUNK)
                def _():
                    _start(j + 1, 1 - b)

                if do_adj:
                    _adj_step(2 * j, 0)
                    _adj_step(2 * j + 1, 1)

                pltpu.make_async_copy(h_hbm.at[srcs.at[j]], rows2.at[b],
                                      gsem.at[b]).wait()
                pltpu.make_async_copy(
                    ee_hbm.at[pl.ds(ebase + j * _ECH, _ECH), :],
                    ee2.at[b], esem.at[b]).wait()

                @plsc.parallel_loop(0, _ECH, unroll=2)
                def _row(r):
                    for cb in range(HID // L):
                        sl = pl.ds(cb * L, L)
                        rows2[b, r, sl] = jnp.maximum(
                            rows2[b, r, sl] + ee2[b, r, sl], 0.0)

                pltpu.async_copy(rows2.at[b], acc.at[dsts.at[j]],
                                 ssem.at[b], add=True)

        for b in range(2):
            j = _NCHUNK - 2 + b
            pltpu.make_async_copy(rows2.at[b], acc.at[dsts.at[j]],
                                  ssem.at[b]).wait()
        if do_adj:
            pltpu.sync_copy(win, adj_hbm.at[pl.ds(wbase, _WIN)])
        plsc.subcore_barrier()
        pltpu.sync_copy(acc.at[pl.ds(s * _RPT, _RPT)],
                        out_hbm.at[c, pl.ds(s * _RPT, _RPT)])

    return _kernel


_gine_aggr_adj_kernel = _make_aggr(True)
_gine_aggr_kernel = _make_aggr(False)


def _node_body(h_ref, p_ref, eps_ref, w1_ref, b1_ref, w2_ref, b2_ref,
               g_ref, be_ref, o_ref):
    z = (1.0 + eps_ref[0, 0]) * h_ref[...] + p_ref[0] + p_ref[1]
    z = jnp.maximum(jnp.dot(z, w1_ref[...],
                            preferred_element_type=jnp.float32) + b1_ref[...], 0.0)
    z = jnp.dot(z, w2_ref[...], preferred_element_type=jnp.float32) + b2_ref[...]
    mu = jnp.mean(z, axis=0, keepdims=True)
    var = jnp.mean(jnp.square(z - mu), axis=0, keepdims=True)
    z = (z - mu) * jax.lax.rsqrt(var + 1e-5) * g_ref[...] + be_ref[...]
    o_ref[...] = jnp.maximum(z, 0.0)


def _node_update(h, partials, lp):
    return pl.pallas_call(
        _node_body,
        out_shape=jax.ShapeDtypeStruct((N_NODES, HID), jnp.float32),
    )(h, partials, lp['eps'][None, None],
      lp['nn_W1'], lp['nn_b1'][None, :], lp['nn_W2'], lp['nn_b2'][None, :],
      lp['bn_gamma'][None, :], lp['bn_beta'][None, :])


def _node2_body(h_ref, p_ref, eps_ref, w1_ref, b1_ref, w2_ref, b2_ref,
                g_ref, be_ref, wa_ref, wb_ref, mb1_ref, a_ref, bm_ref):
    z = (1.0 + eps_ref[0, 0]) * h_ref[...] + p_ref[0] + p_ref[1]
    z = jnp.maximum(jnp.dot(z, w1_ref[...],
                            preferred_element_type=jnp.float32) + b1_ref[...], 0.0)
    z = jnp.dot(z, w2_ref[...], preferred_element_type=jnp.float32) + b2_ref[...]
    mu = jnp.mean(z, axis=0, keepdims=True)
    var = jnp.mean(jnp.square(z - mu), axis=0, keepdims=True)
    z = (z - mu) * jax.lax.rsqrt(var + 1e-5) * g_ref[...] + be_ref[...]
    h2 = jnp.maximum(z, 0.0)
    a_ref[...] = jnp.dot(h2, wa_ref[...],
                         preferred_element_type=jnp.float32) + mb1_ref[...]
    bm_ref[...] = jnp.dot(h2, wb_ref[...], preferred_element_type=jnp.float32)


def _node2_and_pair_ab(h, partials, lp, params):
    w1 = params['mlp_W1']
    return pl.pallas_call(
        _node2_body,
        out_shape=[jax.ShapeDtypeStruct((N_NODES, HID), jnp.float32)] * 2,
    )(h, partials, lp['eps'][None, None],
      lp['nn_W1'], lp['nn_b1'][None, :], lp['nn_W2'], lp['nn_b2'][None, :],
      lp['bn_gamma'][None, :], lp['bn_beta'][None, :],
      w1[:HID], w1[HID:2 * HID], params['mlp_b1'][None, :])


_GPB = 4


def _pair_body(a_ref, b_ref, adj_ref, w_ref, w2_ref, b2_ref, o_ref):
    w = w_ref[0][None, None, :].astype(jnp.bfloat16)
    w2 = w2_ref[...].astype(jnp.bfloat16)
    for g in range(_GPB):
        a = a_ref[g].astype(jnp.bfloat16)
        b = b_ref[g].astype(jnp.bfloat16)
        adj = adj_ref[g].astype(jnp.bfloat16)
        t = a[:, None, :] + b[None, :, :] + adj[:, :, None] * w
        t = jnp.maximum(t, 0.0).reshape(NPG * NPG, HID)
        o_ref[g] = (jnp.dot(t, w2, preferred_element_type=jnp.float32)
                    + b2_ref[0, 0]).reshape(NPG, NPG)


def _pair_stage(a, bmat, adjflat, params):
    out = pl.pallas_call(
        _pair_body,
        grid=(B // _GPB,),
        in_specs=[
            pl.BlockSpec((_GPB, NPG, HID), lambda i: (i, 0, 0)),
            pl.BlockSpec((_GPB, NPG, HID), lambda i: (i, 0, 0)),
            pl.BlockSpec((_GPB, NPG, NPG), lambda i: (i, 0, 0)),
            pl.BlockSpec((1, HID), lambda i: (0, 0)),
            pl.BlockSpec((HID, 1), lambda i: (0, 0)),
            pl.BlockSpec((1, 1), lambda i: (0, 0)),
        ],
        out_specs=pl.BlockSpec((_GPB, NPG, NPG), lambda i: (i, 0, 0)),
        out_shape=jax.ShapeDtypeStruct((B, NPG, NPG), jnp.float32),
    )(a.reshape(B, NPG, HID), bmat.reshape(B, NPG, HID),
      adjflat.reshape(B, NPG, NPG),
      params['mlp_W1'][2 * HID][None, :], params['mlp_W2'],
      params['mlp_b2'][None, :])
    return out.reshape(B * NPG * NPG, 1)


@jax.jit
def kernel(x, edge_index, edge_attr, params):
    lp1, lp2 = params['layers']
    zeros = jnp.zeros((N_NODES, HID), jnp.float32)
    ei4 = edge_index.reshape(2, NW, _NCHUNK, _ECH)
    ee1, h0 = _edge1_and_atom(edge_attr, x, params)
    partials1, adjflat = _gine_aggr_adj_kernel(h0, ee1, ei4, zeros)
    ee2 = _edge2_mlp(edge_attr, params)
    h1 = _node_update(h0, partials1, lp1)
    partials2, = _gine_aggr_kernel(h1, ee2, ei4, zeros)
    a, bmat = _node2_and_pair_ab(h1, partials2, lp2, params)
    return _pair_stage(a, bmat, adjflat, params)


# speedup vs baseline: 1.2638x; 1.0374x over previous
import functools

import jax
import jax.numpy as jnp
from jax import lax
from jax.experimental import pallas as pl
from jax.experimental.pallas import tpu as pltpu
from jax.experimental.pallas import tpu_sc as plsc

N_NODES = 4096
N_EDGES = 65536
B = 64
NPG = 64
D_IN = 128
D_EDGE = 16
HID = 128

NC = 2
NS = 16
NW = NC * NS
L = 16


_EB = 2048
_XB = N_NODES // (N_EDGES // _EB)


def _edge1_body(ea_ref, bw_ref, bb_ref, w1_ref, b1_ref, w2_ref, b2_ref,
                x_ref, aw_ref, ab_ref, ee_ref, h0_ref):
    e0 = jnp.dot(ea_ref[...], bw_ref[...],
                 preferred_element_type=jnp.float32) + bb_ref[...]
    t = jnp.maximum(jnp.dot(e0, w1_ref[...],
                            preferred_element_type=jnp.float32) + b1_ref[...], 0.0)
    ee_ref[...] = jnp.dot(t, w2_ref[...],
                          preferred_element_type=jnp.float32) + b2_ref[...]
    h0_ref[...] = jnp.dot(x_ref[...], aw_ref[...],
                          preferred_element_type=jnp.float32) + ab_ref[...]


def _edge1_and_atom(edge_attr, x, params):
    lp1 = params['layers'][0]
    full = lambda shape: pl.BlockSpec(shape, lambda i: (0,) * len(shape))
    row = lambda d: pl.BlockSpec((_EB, d), lambda i: (i, 0))
    return pl.pallas_call(
        _edge1_body,
        grid=(N_EDGES // _EB,),
        in_specs=[
            row(D_EDGE),
            full((D_EDGE, HID)), full((1, HID)),
            full((HID, HID)), full((1, HID)), full((HID, HID)), full((1, HID)),
            pl.BlockSpec((_XB, D_IN), lambda i: (i, 0)),
            full((D_IN, HID)), full((1, HID)),
        ],
        out_specs=[row(HID), pl.BlockSpec((_XB, HID), lambda i: (i, 0))],
        out_shape=[jax.ShapeDtypeStruct((N_EDGES, HID), jnp.float32),
                   jax.ShapeDtypeStruct((N_NODES, HID), jnp.float32)],
    )(edge_attr,
      params['bond_W'], params['bond_b'][None, :],
      lp1['be_W1'], lp1['be_b1'][None, :],
      lp1['be_W2'], lp1['be_b2'][None, :],
      x, params['atom_W'], params['atom_b'][None, :])


def _edge2_body(ea_ref, bw_ref, bb_ref, w1_ref, b1_ref, w2_ref, b2_ref,
                ee_ref):
    e0 = jnp.dot(ea_ref[...], bw_ref[...],
                 preferred_element_type=jnp.float32) + bb_ref[...]
    t = jnp.maximum(jnp.dot(e0, w1_ref[...],
                            preferred_element_type=jnp.float32) + b1_ref[...], 0.0)
    ee_ref[...] = jnp.dot(t, w2_ref[...],
                          preferred_element_type=jnp.float32) + b2_ref[...]


def _edge2_mlp(edge_attr, params):
    lp2 = params['layers'][1]
    full = lambda shape: pl.BlockSpec(shape, lambda i: (0,) * len(shape))
    row = lambda d: pl.BlockSpec((_EB, d), lambda i: (i, 0))
    return pl.pallas_call(
        _edge2_body,
        grid=(N_EDGES // _EB,),
        in_specs=[
            row(D_EDGE),
            full((D_EDGE, HID)), full((1, HID)),
            full((HID, HID)), full((1, HID)), full((HID, HID)), full((1, HID)),
        ],
        out_specs=row(HID),
        out_shape=jax.ShapeDtypeStruct((N_EDGES, HID), jnp.float32),
    )(edge_attr, params['bond_W'], params['bond_b'][None, :],
      lp2['be_W1'], lp2['be_b1'][None, :],
      lp2['be_W2'], lp2['be_b2'][None, :])


_ECH = 128
_EPW = N_EDGES // NW
_NCHUNK = _EPW // _ECH
_RPT = N_NODES // NS
_NPAIR = B * NPG * NPG
_WIN = _NPAIR // NW
_ACH = 2048

_sc_mesh = plsc.VectorSubcoreMesh(core_axis_name="c", subcore_axis_name="s")


def _make_aggr(do_adj):
    out_type = [jax.ShapeDtypeStruct((NC, N_NODES, HID), jnp.float32)]
    scratch = [
        pltpu.VMEM((_NCHUNK, _ECH), jnp.int32),
        pltpu.VMEM((_NCHUNK, _ECH), jnp.int32),
        pltpu.VMEM((2, _ECH, HID), jnp.float32),
        pltpu.VMEM((2, _ECH, HID), jnp.float32),
        pltpu.VMEM_SHARED((N_NODES, HID), jnp.float32),
        pltpu.SemaphoreType.DMA((2,)),
        pltpu.SemaphoreType.DMA((2,)),
        pltpu.SemaphoreType.DMA((2,)),
    ]
    if do_adj:
        out_type.append(jax.ShapeDtypeStruct((_NPAIR,), jnp.float32))
        scratch += [
            pltpu.VMEM((_WIN,), jnp.float32),
            pltpu.VMEM((2, _NCHUNK, _ECH), jnp.int32),
            pltpu.VMEM((2, _NCHUNK, _ECH), jnp.int32),
            pltpu.SemaphoreType.DMA((2,)),
            pltpu.SemaphoreType.DMA((2,)),
        ]

    @functools.partial(
        pl.kernel,
        out_type=out_type,
        mesh=_sc_mesh,
        scratch_types=scratch,
        compiler_params=pltpu.CompilerParams(needs_layout_passes=False),
    )
    def _kernel(h_hbm, ee_hbm, ei_hbm, zeros_hbm, out_hbm, *rest):
        if do_adj:
            (adj_hbm, srcs, dsts, rows2, ee2, acc, gsem, esem, ssem,
             win, asv, adv, assem, adsem) = rest
        else:
            srcs, dsts, rows2, ee2, acc, gsem, esem, ssem = rest
        c = lax.axis_index("c")
        s = lax.axis_index("s")
        wid = c * NS + s
        ebase = wid * _EPW
        pltpu.sync_copy(zeros_hbm.at[pl.ds(s * _RPT, _RPT)],
                        acc.at[pl.ds(s * _RPT, _RPT)])
        pltpu.sync_copy(ei_hbm.at[0, wid], srcs)
        pltpu.sync_copy(ei_hbm.at[1, wid], dsts)

        def _start(j, b):
            pltpu.async_copy(h_hbm.at[srcs.at[j]], rows2.at[b], gsem.at[b])
            pltpu.async_copy(ee_hbm.at[pl.ds(ebase + j * _ECH, _ECH), :],
                             ee2.at[b], esem.at[b])

        if do_adj:
            wbase = wid * _WIN
            ones = jnp.full((L,), 1.0, jnp.float32)

            def _adj_start(t, ab):
                pltpu.async_copy(ei_hbm.at[0, t], asv.at[ab], assem.at[ab])
                pltpu.async_copy(ei_hbm.at[1, t], adv.at[ab], adsem.at[ab])

            def _adj_step(t, ab):
                @pl.when(t + 1 < NW)
                def _():
                    _adj_start(t + 1, 1 - ab)

                pltpu.make_async_copy(ei_hbm.at[0, t], asv.at[ab],
                                      assem.at[ab]).wait()
                pltpu.make_async_copy(ei_hbm.at[1, t], adv.at[ab],
                                      adsem.at[ab]).wait()

                @plsc.parallel_loop(0, _NCHUNK, unroll=2)
                def _inner(rr):
                    for cb in range(_ECH // L):
                        sl = pl.ds(cb * L, L)
                        sv = asv[ab, rr, sl]
                        dv = adv[ab, rr, sl]
                        pos = sv * NPG + (dv & (NPG - 1)) - wbase
                        m = ((sv >> 6) == (dv >> 6)) & (pos >= 0) & (pos < _WIN)
                        plsc.store_scatter(win, [pos], ones, mask=m)

            _adj_start(0, 0)

            @plsc.parallel_loop(0, _WIN // L, unroll=8)
            def _z(i):
                win[pl.ds(i * L, L)] = jnp.zeros((L,), jnp.float32)

        plsc.subcore_barrier()
        _start(0, 0)

        @pl.loop(0, _NCHUNK, step=2)
        def _chunk(j0):
            for b in range(2):
                j = j0 + b

                @pl.when((j >= 1) & (j + 1 < _NCHUNK))
                def _():
                    pltpu.make_async_copy(rows2.at[1 - b],
                                          acc.at[dsts.at[j - 1]],
                                          ssem.at[1 - b]).wait()

                @pl.when(j + 1 < _NCHUNK)
                def _():
                    _start(j + 1, 1 - b)

                if do_adj:
                    _adj_step(2 * j, 0)
                    _adj_step(2 * j + 1, 1)

                pltpu.make_async_copy(h_hbm.at[srcs.at[j]], rows2.at[b],
                                      gsem.at[b]).wait()
                pltpu.make_async_copy(
                    ee_hbm.at[pl.ds(ebase + j * _ECH, _ECH), :],
                    ee2.at[b], esem.at[b]).wait()

                @plsc.parallel_loop(0, _ECH, unroll=2)
                def _row(r):
                    for cb in range(HID // L):
                        sl = pl.ds(cb * L, L)
                        rows2[b, r, sl] = jnp.maximum(
                            rows2[b, r, sl] + ee2[b, r, sl], 0.0)

                pltpu.async_copy(rows2.at[b], acc.at[dsts.at[j]],
                                 ssem.at[b], add=True)

        for b in range(2):
            j = _NCHUNK - 2 + b
            pltpu.make_async_copy(rows2.at[b], acc.at[dsts.at[j]],
                                  ssem.at[b]).wait()
        if do_adj:
            pltpu.sync_copy(win, adj_hbm.at[pl.ds(wbase, _WIN)])
        plsc.subcore_barrier()
        pltpu.sync_copy(acc.at[pl.ds(s * _RPT, _RPT)],
                        out_hbm.at[c, pl.ds(s * _RPT, _RPT)])

    return _kernel


_gine_aggr_adj_kernel = _make_aggr(True)
_gine_aggr_kernel = _make_aggr(False)


def _node_body(h_ref, p_ref, eps_ref, w1_ref, b1_ref, w2_ref, b2_ref,
               g_ref, be_ref, o_ref):
    z = (1.0 + eps_ref[0, 0]) * h_ref[...] + p_ref[0] + p_ref[1]
    z = jnp.maximum(jnp.dot(z, w1_ref[...],
                            preferred_element_type=jnp.float32) + b1_ref[...], 0.0)
    z = jnp.dot(z, w2_ref[...], preferred_element_type=jnp.float32) + b2_ref[...]
    mu = jnp.mean(z, axis=0, keepdims=True)
    var = jnp.mean(jnp.square(z - mu), axis=0, keepdims=True)
    z = (z - mu) * jax.lax.rsqrt(var + 1e-5) * g_ref[...] + be_ref[...]
    o_ref[...] = jnp.maximum(z, 0.0)


def _node_update(h, partials, lp):
    return pl.pallas_call(
        _node_body,
        out_shape=jax.ShapeDtypeStruct((N_NODES, HID), jnp.float32),
    )(h, partials, lp['eps'][None, None],
      lp['nn_W1'], lp['nn_b1'][None, :], lp['nn_W2'], lp['nn_b2'][None, :],
      lp['bn_gamma'][None, :], lp['bn_beta'][None, :])


def _node2_body(h_ref, p_ref, eps_ref, w1_ref, b1_ref, w2_ref, b2_ref,
                g_ref, be_ref, wa_ref, wb_ref, mb1_ref, a_ref, bm_ref):
    z = (1.0 + eps_ref[0, 0]) * h_ref[...] + p_ref[0] + p_ref[1]
    z = jnp.maximum(jnp.dot(z, w1_ref[...],
                            preferred_element_type=jnp.float32) + b1_ref[...], 0.0)
    z = jnp.dot(z, w2_ref[...], preferred_element_type=jnp.float32) + b2_ref[...]
    mu = jnp.mean(z, axis=0, keepdims=True)
    var = jnp.mean(jnp.square(z - mu), axis=0, keepdims=True)
    z = (z - mu) * jax.lax.rsqrt(var + 1e-5) * g_ref[...] + be_ref[...]
    h2 = jnp.maximum(z, 0.0)
    a_ref[...] = jnp.dot(h2, wa_ref[...],
                         preferred_element_type=jnp.float32) + mb1_ref[...]
    bm_ref[...] = jnp.dot(h2, wb_ref[...], preferred_element_type=jnp.float32)


def _node2_and_pair_ab(h, partials, lp, params):
    w1 = params['mlp_W1']
    return pl.pallas_call(
        _node2_body,
        out_shape=[jax.ShapeDtypeStruct((N_NODES, HID), jnp.float32)] * 2,
    )(h, partials, lp['eps'][None, None],
      lp['nn_W1'], lp['nn_b1'][None, :], lp['nn_W2'], lp['nn_b2'][None, :],
      lp['bn_gamma'][None, :], lp['bn_beta'][None, :],
      w1[:HID], w1[HID:2 * HID], params['mlp_b1'][None, :])


_GPB = 4


def _pair_body(a_ref, b_ref, adj_ref, w_ref, w2_ref, b2_ref, o_ref):
    w = w_ref[0][None, None, :].astype(jnp.bfloat16)
    w2 = w2_ref[...].astype(jnp.bfloat16)
    for g in range(_GPB):
        a = a_ref[g].astype(jnp.bfloat16)
        b = b_ref[g].astype(jnp.bfloat16)
        adj = adj_ref[g].astype(jnp.bfloat16)
        t = a[:, None, :] + b[None, :, :] + adj[:, :, None] * w
        t = jnp.maximum(t, 0.0).reshape(NPG * NPG, HID)
        o_ref[g] = (jnp.dot(t, w2, preferred_element_type=jnp.float32)
                    + b2_ref[0, 0]).reshape(NPG, NPG)


def _pair_stage(a, bmat, adjflat, params):
    out = pl.pallas_call(
        _pair_body,
        grid=(B // _GPB,),
        in_specs=[
            pl.BlockSpec((_GPB, NPG, HID), lambda i: (i, 0, 0)),
            pl.BlockSpec((_GPB, NPG, HID), lambda i: (i, 0, 0)),
            pl.BlockSpec((_GPB, NPG, NPG), lambda i: (i, 0, 0)),
            pl.BlockSpec((1, HID), lambda i: (0, 0)),
            pl.BlockSpec((HID, 1), lambda i: (0, 0)),
            pl.BlockSpec((1, 1), lambda i: (0, 0)),
        ],
        out_specs=pl.BlockSpec((_GPB, NPG, NPG), lambda i: (i, 0, 0)),
        out_shape=jax.ShapeDtypeStruct((B, NPG, NPG), jnp.float32),
    )(a.reshape(B, NPG, HID), bmat.reshape(B, NPG, HID),
      adjflat.reshape(B, NPG, NPG),
      params['mlp_W1'][2 * HID][None, :], params['mlp_W2'],
      params['mlp_b2'][None, :])
    return out.reshape(B * NPG * NPG, 1)


@jax.jit
def kernel(x, edge_index, edge_attr, params):
    lp1, lp2 = params['layers']
    zeros = jnp.zeros((N_NODES, HID), jnp.float32)
    ei4 = edge_index.reshape(2, NW, _NCHUNK, _ECH)
    ee1, h0 = _edge1_and_atom(edge_attr, x, params)
    partials1, adjflat = _gine_aggr_adj_kernel(h0, ee1, ei4, zeros)
    ee2 = _edge2_mlp(edge_attr, params)
    h1 = _node_update(h0, partials1, lp1)
    partials2, = _gine_aggr_kernel(h1, ee2, ei4, zeros)
    a, bmat = _node2_and_pair_ab(h1, partials2, lp2, params)
    return _pair_stage(a, bmat, adjflat, params)


# speedup vs baseline: 1.2651x; 1.0010x over previous
import functools

import jax
import jax.numpy as jnp
from jax import lax
from jax.experimental import pallas as pl
from jax.experimental.pallas import tpu as pltpu
from jax.experimental.pallas import tpu_sc as plsc

N_NODES = 4096
N_EDGES = 65536
B = 64
NPG = 64
D_IN = 128
D_EDGE = 16
HID = 128

NC = 2
NS = 16
NW = NC * NS
L = 16


_EB = 2048
_XB = N_NODES // (N_EDGES // _EB)


def _edge1_body(ea_ref, bw_ref, bb_ref, w1_ref, b1_ref, w2_ref, b2_ref,
                x_ref, aw_ref, ab_ref, ee_ref, h0_ref):
    e0 = jnp.dot(ea_ref[...], bw_ref[...],
                 preferred_element_type=jnp.float32) + bb_ref[...]
    t = jnp.maximum(jnp.dot(e0, w1_ref[...],
                            preferred_element_type=jnp.float32) + b1_ref[...], 0.0)
    ee_ref[...] = jnp.dot(t, w2_ref[...],
                          preferred_element_type=jnp.float32) + b2_ref[...]
    h0_ref[...] = jnp.dot(x_ref[...], aw_ref[...],
                          preferred_element_type=jnp.float32) + ab_ref[...]


def _edge1_and_atom(edge_attr, x, params):
    lp1 = params['layers'][0]
    full = lambda shape: pl.BlockSpec(shape, lambda i: (0,) * len(shape))
    row = lambda d: pl.BlockSpec((_EB, d), lambda i: (i, 0))
    return pl.pallas_call(
        _edge1_body,
        grid=(N_EDGES // _EB,),
        in_specs=[
            row(D_EDGE),
            full((D_EDGE, HID)), full((1, HID)),
            full((HID, HID)), full((1, HID)), full((HID, HID)), full((1, HID)),
            pl.BlockSpec((_XB, D_IN), lambda i: (i, 0)),
            full((D_IN, HID)), full((1, HID)),
        ],
        out_specs=[row(HID), pl.BlockSpec((_XB, HID), lambda i: (i, 0))],
        out_shape=[jax.ShapeDtypeStruct((N_EDGES, HID), jnp.float32),
                   jax.ShapeDtypeStruct((N_NODES, HID), jnp.float32)],
    )(edge_attr,
      params['bond_W'], params['bond_b'][None, :],
      lp1['be_W1'], lp1['be_b1'][None, :],
      lp1['be_W2'], lp1['be_b2'][None, :],
      x, params['atom_W'], params['atom_b'][None, :])


def _edge2_body(ea_ref, bw_ref, bb_ref, w1_ref, b1_ref, w2_ref, b2_ref,
                ee_ref):
    e0 = jnp.dot(ea_ref[...], bw_ref[...],
                 preferred_element_type=jnp.float32) + bb_ref[...]
    t = jnp.maximum(jnp.dot(e0, w1_ref[...],
                            preferred_element_type=jnp.float32) + b1_ref[...], 0.0)
    ee_ref[...] = jnp.dot(t, w2_ref[...],
                          preferred_element_type=jnp.float32) + b2_ref[...]


def _edge2_mlp(edge_attr, params):
    lp2 = params['layers'][1]
    full = lambda shape: pl.BlockSpec(shape, lambda i: (0,) * len(shape))
    row = lambda d: pl.BlockSpec((_EB, d), lambda i: (i, 0))
    return pl.pallas_call(
        _edge2_body,
        grid=(N_EDGES // _EB,),
        in_specs=[
            row(D_EDGE),
            full((D_EDGE, HID)), full((1, HID)),
            full((HID, HID)), full((1, HID)), full((HID, HID)), full((1, HID)),
        ],
        out_specs=row(HID),
        out_shape=jax.ShapeDtypeStruct((N_EDGES, HID), jnp.float32),
    )(edge_attr, params['bond_W'], params['bond_b'][None, :],
      lp2['be_W1'], lp2['be_b1'][None, :],
      lp2['be_W2'], lp2['be_b2'][None, :])


_ECH = 128
_EPW = N_EDGES // NW
_NCHUNK = _EPW // _ECH
_RPT = N_NODES // NS
_NPAIR = B * NPG * NPG
_WIN = _NPAIR // NW
_ACH = 2048

_sc_mesh = plsc.VectorSubcoreMesh(core_axis_name="c", subcore_axis_name="s")


def _make_aggr(do_adj):
    out_type = [jax.ShapeDtypeStruct((NC, N_NODES, HID), jnp.float32)]
    scratch = [
        pltpu.VMEM((_NCHUNK, _ECH), jnp.int32),
        pltpu.VMEM((_NCHUNK, _ECH), jnp.int32),
        pltpu.VMEM((2, _ECH, HID), jnp.float32),
        pltpu.VMEM((2, _ECH, HID), jnp.float32),
        pltpu.VMEM_SHARED((N_NODES, HID), jnp.float32),
        pltpu.SemaphoreType.DMA((2,)),
        pltpu.SemaphoreType.DMA((2,)),
        pltpu.SemaphoreType.DMA((2,)),
    ]
    if do_adj:
        out_type.append(jax.ShapeDtypeStruct((_NPAIR,), jnp.float32))
        scratch += [
            pltpu.VMEM((_WIN,), jnp.float32),
            pltpu.VMEM((2, _NCHUNK, _ECH), jnp.int32),
            pltpu.VMEM((2, _NCHUNK, _ECH), jnp.int32),
            pltpu.SemaphoreType.DMA((2,)),
            pltpu.SemaphoreType.DMA((2,)),
        ]

    @functools.partial(
        pl.kernel,
        out_type=out_type,
        mesh=_sc_mesh,
        scratch_types=scratch,
        compiler_params=pltpu.CompilerParams(needs_layout_passes=False),
    )
    def _kernel(h_hbm, ee_hbm, ei_hbm, zeros_hbm, out_hbm, *rest):
        if do_adj:
            (adj_hbm, srcs, dsts, rows2, ee2, acc, gsem, esem, ssem,
             win, asv, adv, assem, adsem) = rest
        else:
            srcs, dsts, rows2, ee2, acc, gsem, esem, ssem = rest
        c = lax.axis_index("c")
        s = lax.axis_index("s")
        wid = c * NS + s
        ebase = wid * _EPW
        pltpu.sync_copy(zeros_hbm.at[pl.ds(s * _RPT, _RPT)],
                        acc.at[pl.ds(s * _RPT, _RPT)])
        pltpu.sync_copy(ei_hbm.at[0, wid], srcs)
        pltpu.sync_copy(ei_hbm.at[1, wid], dsts)

        def _start(j, b):
            pltpu.async_copy(h_hbm.at[srcs.at[j]], rows2.at[b], gsem.at[b])
            pltpu.async_copy(ee_hbm.at[pl.ds(ebase + j * _ECH, _ECH), :],
                             ee2.at[b], esem.at[b])

        if do_adj:
            wbase = wid * _WIN
            ones = jnp.full((L,), 1.0, jnp.float32)

            def _adj_start(t, ab):
                pltpu.async_copy(ei_hbm.at[0, t], asv.at[ab], assem.at[ab])
                pltpu.async_copy(ei_hbm.at[1, t], adv.at[ab], adsem.at[ab])

            def _adj_step(t, ab):
                @pl.when(t + 1 < NW)
                def _():
                    _adj_start(t + 1, 1 - ab)

                pltpu.make_async_copy(ei_hbm.at[0, t], asv.at[ab],
                                      assem.at[ab]).wait()
                pltpu.make_async_copy(ei_hbm.at[1, t], adv.at[ab],
                                      adsem.at[ab]).wait()

                @plsc.parallel_loop(0, _NCHUNK, unroll=2)
                def _inner(rr):
                    for cb in range(_ECH // L):
                        sl = pl.ds(cb * L, L)
                        sv = asv[ab, rr, sl]
                        dv = adv[ab, rr, sl]
                        pos = sv * NPG + (dv & (NPG - 1)) - wbase
                        m = ((sv >> 6) == (dv >> 6)) & (pos >= 0) & (pos < _WIN)
                        plsc.store_scatter(win, [pos], ones, mask=m)

            _adj_start(0, 0)

            @plsc.parallel_loop(0, _WIN // L, unroll=8)
            def _z(i):
                win[pl.ds(i * L, L)] = jnp.zeros((L,), jnp.float32)

        plsc.subcore_barrier()
        _start(0, 0)

        @pl.loop(0, _NCHUNK, step=2)
        def _chunk(j0):
            for b in range(2):
                j = j0 + b

                @pl.when((j >= 1) & (j + 1 < _NCHUNK))
                def _():
                    pltpu.make_async_copy(rows2.at[1 - b],
                                          acc.at[dsts.at[j - 1]],
                                          ssem.at[1 - b]).wait()

                @pl.when(j + 1 < _NCHUNK)
                def _():
                    _start(j + 1, 1 - b)

                if do_adj:
                    _adj_step(2 * j, 0)
                    _adj_step(2 * j + 1, 1)

                pltpu.make_async_copy(h_hbm.at[srcs.at[j]], rows2.at[b],
                                      gsem.at[b]).wait()
                pltpu.make_async_copy(
                    ee_hbm.at[pl.ds(ebase + j * _ECH, _ECH), :],
                    ee2.at[b], esem.at[b]).wait()

                @plsc.parallel_loop(0, _ECH, unroll=2)
                def _row(r):
                    for cb in range(HID // L):
                        sl = pl.ds(cb * L, L)
                        rows2[b, r, sl] = jnp.maximum(
                            rows2[b, r, sl] + ee2[b, r, sl], 0.0)

                pltpu.async_copy(rows2.at[b], acc.at[dsts.at[j]],
                                 ssem.at[b], add=True)

        for b in range(2):
            j = _NCHUNK - 2 + b
            pltpu.make_async_copy(rows2.at[b], acc.at[dsts.at[j]],
                                  ssem.at[b]).wait()
        if do_adj:
            pltpu.sync_copy(win, adj_hbm.at[pl.ds(wbase, _WIN)])
        plsc.subcore_barrier()
        pltpu.sync_copy(acc.at[pl.ds(s * _RPT, _RPT)],
                        out_hbm.at[c, pl.ds(s * _RPT, _RPT)])

    return _kernel


_gine_aggr_adj_kernel = _make_aggr(True)
_gine_aggr_kernel = _make_aggr(False)


def _node_body(h_ref, p_ref, eps_ref, w1_ref, b1_ref, w2_ref, b2_ref,
               g_ref, be_ref, o_ref):
    z = (1.0 + eps_ref[0, 0]) * h_ref[...] + p_ref[0] + p_ref[1]
    z = jnp.maximum(jnp.dot(z, w1_ref[...],
                            preferred_element_type=jnp.float32) + b1_ref[...], 0.0)
    z = jnp.dot(z, w2_ref[...], preferred_element_type=jnp.float32) + b2_ref[...]
    mu = jnp.mean(z, axis=0, keepdims=True)
    var = jnp.mean(jnp.square(z - mu), axis=0, keepdims=True)
    z = (z - mu) * jax.lax.rsqrt(var + 1e-5) * g_ref[...] + be_ref[...]
    o_ref[...] = jnp.maximum(z, 0.0)


def _node_update(h, partials, lp):
    return pl.pallas_call(
        _node_body,
        out_shape=jax.ShapeDtypeStruct((N_NODES, HID), jnp.float32),
    )(h, partials, lp['eps'][None, None],
      lp['nn_W1'], lp['nn_b1'][None, :], lp['nn_W2'], lp['nn_b2'][None, :],
      lp['bn_gamma'][None, :], lp['bn_beta'][None, :])


def _node2_body(h_ref, p_ref, eps_ref, w1_ref, b1_ref, w2_ref, b2_ref,
                g_ref, be_ref, wa_ref, wb_ref, mb1_ref, a_ref, bm_ref):
    z = (1.0 + eps_ref[0, 0]) * h_ref[...] + p_ref[0] + p_ref[1]
    z = jnp.maximum(jnp.dot(z, w1_ref[...],
                            preferred_element_type=jnp.float32) + b1_ref[...], 0.0)
    z = jnp.dot(z, w2_ref[...], preferred_element_type=jnp.float32) + b2_ref[...]
    mu = jnp.mean(z, axis=0, keepdims=True)
    var = jnp.mean(jnp.square(z - mu), axis=0, keepdims=True)
    z = (z - mu) * jax.lax.rsqrt(var + 1e-5) * g_ref[...] + be_ref[...]
    h2 = jnp.maximum(z, 0.0)
    a_ref[...] = jnp.dot(h2, wa_ref[...],
                         preferred_element_type=jnp.float32) + mb1_ref[...]
    bm_ref[...] = jnp.dot(h2, wb_ref[...], preferred_element_type=jnp.float32)


def _node2_and_pair_ab(h, partials, lp, params):
    w1 = params['mlp_W1']
    return pl.pallas_call(
        _node2_body,
        out_shape=[jax.ShapeDtypeStruct((N_NODES, HID), jnp.float32)] * 2,
    )(h, partials, lp['eps'][None, None],
      lp['nn_W1'], lp['nn_b1'][None, :], lp['nn_W2'], lp['nn_b2'][None, :],
      lp['bn_gamma'][None, :], lp['bn_beta'][None, :],
      w1[:HID], w1[HID:2 * HID], params['mlp_b1'][None, :])


_GPB = 8


def _pair_body(a_ref, b_ref, adj_ref, w_ref, w2_ref, b2_ref, o_ref):
    w = w_ref[0][None, None, :].astype(jnp.bfloat16)
    w2 = w2_ref[...].astype(jnp.bfloat16)
    for g in range(_GPB):
        a = a_ref[g].astype(jnp.bfloat16)
        b = b_ref[g].astype(jnp.bfloat16)
        adj = adj_ref[g].astype(jnp.bfloat16)
        t = a[:, None, :] + b[None, :, :] + adj[:, :, None] * w
        t = jnp.maximum(t, 0.0).reshape(NPG * NPG, HID)
        o_ref[g] = (jnp.dot(t, w2, preferred_element_type=jnp.float32)
                    + b2_ref[0, 0]).reshape(NPG, NPG)


def _pair_stage(a, bmat, adjflat, params):
    out = pl.pallas_call(
        _pair_body,
        grid=(B // _GPB,),
        in_specs=[
            pl.BlockSpec((_GPB, NPG, HID), lambda i: (i, 0, 0)),
            pl.BlockSpec((_GPB, NPG, HID), lambda i: (i, 0, 0)),
            pl.BlockSpec((_GPB, NPG, NPG), lambda i: (i, 0, 0)),
            pl.BlockSpec((1, HID), lambda i: (0, 0)),
            pl.BlockSpec((HID, 1), lambda i: (0, 0)),
            pl.BlockSpec((1, 1), lambda i: (0, 0)),
        ],
        out_specs=pl.BlockSpec((_GPB, NPG, NPG), lambda i: (i, 0, 0)),
        out_shape=jax.ShapeDtypeStruct((B, NPG, NPG), jnp.float32),
    )(a.reshape(B, NPG, HID), bmat.reshape(B, NPG, HID),
      adjflat.reshape(B, NPG, NPG),
      params['mlp_W1'][2 * HID][None, :], params['mlp_W2'],
      params['mlp_b2'][None, :])
    return out.reshape(B * NPG * NPG, 1)


@jax.jit
def kernel(x, edge_index, edge_attr, params):
    lp1, lp2 = params['layers']
    zeros = jnp.zeros((N_NODES, HID), jnp.float32)
    ei4 = edge_index.reshape(2, NW, _NCHUNK, _ECH)
    ee1, h0 = _edge1_and_atom(edge_attr, x, params)
    partials1, adjflat = _gine_aggr_adj_kernel(h0, ee1, ei4, zeros)
    ee2 = _edge2_mlp(edge_attr, params)
    h1 = _node_update(h0, partials1, lp1)
    partials2, = _gine_aggr_kernel(h1, ee2, ei4, zeros)
    a, bmat = _node2_and_pair_ab(h1, partials2, lp2, params)
    return _pair_stage(a, bmat, adjflat, params)
